# baseline (speedup 1.0000x reference)
import functools

import numpy as np
import jax
import jax.numpy as jnp
from jax import lax
from jax.experimental import pallas as pl
from jax.experimental.pallas import tpu as pltpu
from jax.experimental.pallas import tpu_sc as plsc

_LC = 5.0
_LN = 0.5
_ANCHORS = (
    (1.3221, 1.73145),
    (3.19275, 4.00944),
    (5.05587, 8.09892),
    (9.47112, 4.84053),
    (11.2364, 10.0071),
)
_A = 5
_S = 13
_G = 30
_B = 64
_NWK = 16
_SPW = _B // _NWK
_CELLS = _S * _S
_BPG = 32
_NROWS = _B * _CELLS


def _build_wsel():
    w = np.zeros((425, 128), np.float32)
    for a in range(_A):
        for k in range(5):
            w[85 * a + k, 5 * a + k] = 1.0
    return w


_WSEL = _build_wsel()


def _sigmoid(x):
    return 1.0 / (1.0 + jnp.exp(-x))


def _sc_sqrt(x):
    i = plsc.bitcast(x, jnp.int32)
    y = plsc.bitcast((i >> 1) + 0x1FBD1DF5, jnp.float32)
    for _ in range(3):
        y = 0.5 * (y + x / y)
    return y


def _prep_kernel(pred_ref, w_ref, tab_ref, sall_ref):
    i = pl.program_id(0)

    @pl.when(i == 0)
    def _init():
        sall_ref[...] = jnp.zeros((1, 16), jnp.float32)

    x = pred_ref[...].reshape(_BPG * _CELLS, 425)
    tab = jnp.dot(x, w_ref[...], preferred_element_type=jnp.float32)
    tab_ref[...] = tab

    col = jax.lax.broadcasted_iota(jnp.int32, (1, 128), 1)
    m = ((col < 5 * _A) & (col % 5 == 4)).astype(jnp.float32)
    conf = _sigmoid(tab)
    lane0 = (jax.lax.broadcasted_iota(jnp.int32, (1, 16), 1) == 0).astype(jnp.float32)
    sall_ref[...] += jnp.sum(conf * conf * m) * lane0


def _sc_body(tab_ref, tgt_ref, sall_ref, out_ref, tgt_v, idxb_v, rowsb_v, cell_v, obuf_v, shared_v, allp_v, sall_v, sem):
    wid = lax.axis_index("s")
    iota16 = lax.iota(jnp.int32, 16)

    pltpu.sync_copy(tgt_ref.at[pl.ds(_SPW * wid, _SPW)], tgt_v)

    per_vreg = []
    for s in range(_SPW):
        svec = jnp.full((16,), s, jnp.int32)
        sample = []
        for v in range(2):
            t16 = iota16 + 16 * v
            valid = t16 < _G
            tcl = jnp.minimum(t16, _G - 1)
            gx = plsc.load_gather(tgt_v, [svec, tcl * 5 + 0])
            gy = plsc.load_gather(tgt_v, [svec, tcl * 5 + 1])
            gw = plsc.load_gather(tgt_v, [svec, tcl * 5 + 2])
            gh = plsc.load_gather(tgt_v, [svec, tcl * 5 + 3])
            gt_w = gw * float(_S)
            gt_h = gh * float(_S)

            best_iou = jnp.full((16,), -1.0, jnp.float32)
            best_a = jnp.zeros((16,), jnp.int32)
            aw_g = jnp.zeros((16,), jnp.float32)
            ah_g = jnp.zeros((16,), jnp.float32)
            for a, (aw, ah) in enumerate(_ANCHORS):
                inter = jnp.minimum(gt_w, aw) * jnp.minimum(gt_h, ah)
                union = gt_w * gt_h + aw * ah - inter
                iou_a = jnp.where(union > 0, inter / jnp.where(union > 0, union, 1.0), 0.0)
                upd = iou_a > best_iou
                best_iou = jnp.where(upd, iou_a, best_iou)
                best_a = jnp.where(upd, a, best_a)
                aw_g = jnp.where(upd, aw, aw_g)
                ah_g = jnp.where(upd, ah, ah_g)

            gi = jnp.clip((gx * float(_S)).astype(jnp.int32), 0, _S - 1)
            gj = jnp.clip((gy * float(_S)).astype(jnp.int32), 0, _S - 1)
            row169 = gj * _S + gi

            slot = (s * 2 + v) * 16 + iota16
            plsc.store_scatter(idxb_v, [slot], (_SPW * wid + s) * _CELLS + row169)

            cell = jnp.where(valid, best_a * _CELLS + row169, -1)
            sample.append(
                dict(valid=valid, slot=slot, best_a=best_a, gt_w=gt_w, gt_h=gt_h,
                     gx=gx, gy=gy, gi=gi, gj=gj, aw_g=aw_g, ah_g=ah_g, cell=cell)
            )

        c0, c1 = sample[0]["cell"], sample[1]["cell"]
        cell_v[pl.ds(0, 16)] = c0
        cell_v[pl.ds(16, 16)] = c1
        dup0 = jnp.zeros((16,), jnp.bool_)
        dup1 = jnp.zeros((16,), jnp.bool_)
        for sh in range(1, 16):
            ids = jnp.minimum(iota16 + sh, 15)
            ok = iota16 + sh <= 15
            dup0 = dup0 | (ok & (plsc.load_gather(cell_v, [ids]) == c0))
            dup1 = dup1 | (ok & (plsc.load_gather(cell_v, [ids + 16]) == c1))
        for r in range(16):
            rot = plsc.load_gather(cell_v, [(iota16 + r) % 16 + 16])
            dup0 = dup0 | (rot == c0)
        sample[0]["kept"] = sample[0]["valid"] & jnp.logical_not(dup0)
        sample[1]["kept"] = sample[1]["valid"] & jnp.logical_not(dup1)
        per_vreg.append(sample)

    pltpu.async_copy(tab_ref.at[idxb_v], rowsb_v, sem).wait()

    acc_coord = jnp.zeros(16, jnp.float32)
    acc_obj = jnp.zeros(16, jnp.float32)
    acc_noobj = jnp.zeros(16, jnp.float32)
    for s in range(_SPW):
        for v in range(2):
            t = per_vreg[s][v]
            slot = t["slot"]
            col = t["best_a"] * 5

            tx = plsc.load_gather(rowsb_v, [slot, col + 0])
            ty = plsc.load_gather(rowsb_v, [slot, col + 1])
            tw = plsc.load_gather(rowsb_v, [slot, col + 2])
            th = plsc.load_gather(rowsb_v, [slot, col + 3])
            to = plsc.load_gather(rowsb_v, [slot, col + 4])
            pbx = _sigmoid(tx)
            pby = _sigmoid(ty)
            pbw = jnp.exp(tw) * t["aw_g"]
            pbh = jnp.exp(th) * t["ah_g"]
            conf_c = _sigmoid(to)

            gi_f = t["gi"].astype(jnp.float32)
            gj_f = t["gj"].astype(jnp.float32)
            gt_w, gt_h = t["gt_w"], t["gt_h"]
            cx_gt = t["gx"] * float(_S)
            cy_gt = t["gy"] * float(_S)
            ggx = cx_gt - gi_f
            ggy = cy_gt - gj_f
            cx_pr = pbx + gi_f
            cy_pr = pby + gj_f
            iw = jnp.maximum(
                0.0,
                jnp.minimum(cx_gt + gt_w * 0.5, cx_pr + pbw * 0.5)
                - jnp.maximum(cx_gt - gt_w * 0.5, cx_pr - pbw * 0.5),
            )
            ih = jnp.maximum(
                0.0,
                jnp.minimum(cy_gt + gt_h * 0.5, cy_pr + pbh * 0.5)
                - jnp.maximum(cy_gt - gt_h * 0.5, cy_pr - pbh * 0.5),
            )
            inter_a = iw * ih
            union_a = gt_w * gt_h + pbw * pbh - inter_a
            iou = jnp.where(union_a > 0, inter_a / jnp.where(union_a > 0, union_a, 1.0), 0.0)

            dx = pbx - ggx
            dy = pby - ggy
            dw = _sc_sqrt(pbw + 1e-6) - _sc_sqrt(gt_w + 1e-6)
            dh = _sc_sqrt(pbh + 1e-6) - _sc_sqrt(gt_h + 1e-6)
            coord_t = dx * dx + dy * dy + dw * dw + dh * dh
            do = iou - conf_c
            obj_t = do * do
            noobj_t = conf_c * conf_c

            kept = t["kept"]
            acc_coord = acc_coord + jnp.where(kept, coord_t, 0.0)
            acc_obj = acc_obj + jnp.where(kept, obj_t, 0.0)
            acc_noobj = acc_noobj + jnp.where(kept, noobj_t, 0.0)

    part16 = jnp.where(iota16 == 1, jnp.sum(acc_coord), 0.0)
    part16 = part16 + jnp.where(iota16 == 2, jnp.sum(acc_obj), 0.0)
    part16 = part16 + jnp.where(iota16 == 3, jnp.sum(acc_noobj), 0.0)
    obuf_v[...] = part16
    pltpu.sync_copy(obuf_v, shared_v.at[wid])
    plsc.subcore_barrier()

    @pl.when(wid == 0)
    def _final():
        pltpu.sync_copy(shared_v, allp_v)
        pltpu.sync_copy(sall_ref, sall_v)
        comb = jnp.zeros((16,), jnp.float32)
        for r in range(_NWK):
            comb = comb + allp_v[r, :]
        coord_raw = jnp.sum(jnp.where(iota16 == 1, comb, 0.0))
        obj_raw = jnp.sum(jnp.where(iota16 == 2, comb, 0.0))
        noobj_c = jnp.sum(jnp.where(iota16 == 3, comb, 0.0))
        s_all = jnp.sum(jnp.where(iota16 == 0, sall_v[0, :], 0.0))
        coord = _LC * coord_raw
        lnoobj = _LN * (s_all - noobj_c)
        tot = _LC * coord + obj_raw + _LN * lnoobj
        res = jnp.where(iota16 == 0, tot, 0.0)
        res = res + jnp.where(iota16 == 1, coord, 0.0)
        res = res + jnp.where(iota16 == 2, obj_raw + lnoobj, 0.0)
        obuf_v[...] = res
        pltpu.sync_copy(obuf_v, out_ref)


def kernel(predictions, targets, imgs):
    del imgs
    pred3 = predictions.reshape(_B, _CELLS, 425)
    tgt2 = targets.reshape(_B, _G * 5)
    wsel = jnp.asarray(_WSEL)

    scal = jax.ShapeDtypeStruct((1, 16), jnp.float32)
    table, s_all = pl.pallas_call(
        _prep_kernel,
        grid=(_B // _BPG,),
        in_specs=[
            pl.BlockSpec((_BPG, _CELLS, 425), lambda i: (i, 0, 0)),
            pl.BlockSpec((425, 128), lambda i: (0, 0)),
        ],
        out_specs=[
            pl.BlockSpec((_BPG * _CELLS, 128), lambda i: (i, 0)),
            pl.BlockSpec((1, 16), lambda i: (0, 0)),
        ],
        out_shape=[jax.ShapeDtypeStruct((_NROWS, 128), jnp.float32), scal],
    )(pred3, wsel)

    sc_fn = pl.kernel(
        _sc_body,
        out_type=jax.ShapeDtypeStruct((16,), jnp.float32),
        compiler_params=pltpu.CompilerParams(
            use_tc_tiling_on_sc=False, needs_layout_passes=False,
            skip_device_barrier=True,
        ),
        mesh=plsc.VectorSubcoreMesh(
            core_axis_name="c", subcore_axis_name="s", num_cores=1
        ),
        scratch_types=[
            pltpu.VMEM((_SPW, _G * 5), jnp.float32),
            pltpu.VMEM((128,), jnp.int32),
            pltpu.VMEM((128, 128), jnp.float32),
            pltpu.VMEM((32,), jnp.int32),
            pltpu.VMEM((16,), jnp.float32),
            pltpu.VMEM_SHARED((_NWK, 16), jnp.float32),
            pltpu.VMEM((_NWK, 16), jnp.float32),
            pltpu.VMEM((1, 16), jnp.float32),
            pltpu.SemaphoreType.DMA,
        ],
    )
    out = sc_fn(table, tgt2, s_all)
    return (out[0], out[1], out[2])


# speedup vs baseline: 1.0122x; 1.0122x over previous
import functools

import numpy as np
import jax
import jax.numpy as jnp
from jax import lax
from jax.experimental import pallas as pl
from jax.experimental.pallas import tpu as pltpu
from jax.experimental.pallas import tpu_sc as plsc

_LC = 5.0
_LN = 0.5
_ANCHORS = (
    (1.3221, 1.73145),
    (3.19275, 4.00944),
    (5.05587, 8.09892),
    (9.47112, 4.84053),
    (11.2364, 10.0071),
)
_A = 5
_S = 13
_G = 30
_B = 64
_NWK = 16
_SPW = _B // _NWK
_CELLS = _S * _S
_BPG = 16
_NROWS = _B * _CELLS


def _build_wsel():
    w = np.zeros((425, 128), np.float32)
    for a in range(_A):
        for k in range(5):
            w[85 * a + k, 5 * a + k] = 1.0
    return w


_WSEL = _build_wsel()


def _sigmoid(x):
    return 1.0 / (1.0 + jnp.exp(-x))


def _sc_sqrt(x):
    i = plsc.bitcast(x, jnp.int32)
    y = plsc.bitcast((i >> 1) + 0x1FBD1DF5, jnp.float32)
    for _ in range(3):
        y = 0.5 * (y + x / y)
    return y


def _prep_kernel(pred_ref, w_ref, tab_ref, sall_ref):
    i = pl.program_id(0)

    @pl.when(i == 0)
    def _init():
        sall_ref[...] = jnp.zeros((1, 16), jnp.float32)

    x = pred_ref[...].reshape(_BPG * _CELLS, 425)
    tab = jnp.dot(x, w_ref[...], preferred_element_type=jnp.float32)
    tab_ref[...] = tab

    col = jax.lax.broadcasted_iota(jnp.int32, (1, 128), 1)
    m = ((col < 5 * _A) & (col % 5 == 4)).astype(jnp.float32)
    conf = _sigmoid(tab)
    lane0 = (jax.lax.broadcasted_iota(jnp.int32, (1, 16), 1) == 0).astype(jnp.float32)
    sall_ref[...] += jnp.sum(conf * conf * m) * lane0


def _sc_body(tab_ref, tgt_ref, sall_ref, out_ref, tgt_v, idxb_v, rowsb_v, cell_v, obuf_v, shared_v, allp_v, sall_v, sem):
    wid = lax.axis_index("s")
    iota16 = lax.iota(jnp.int32, 16)

    pltpu.sync_copy(tgt_ref.at[pl.ds(_SPW * wid, _SPW)], tgt_v)

    per_vreg = []
    for s in range(_SPW):
        svec = jnp.full((16,), s, jnp.int32)
        sample = []
        for v in range(2):
            t16 = iota16 + 16 * v
            valid = t16 < _G
            tcl = jnp.minimum(t16, _G - 1)
            gx = plsc.load_gather(tgt_v, [svec, tcl * 5 + 0])
            gy = plsc.load_gather(tgt_v, [svec, tcl * 5 + 1])
            gw = plsc.load_gather(tgt_v, [svec, tcl * 5 + 2])
            gh = plsc.load_gather(tgt_v, [svec, tcl * 5 + 3])
            gt_w = gw * float(_S)
            gt_h = gh * float(_S)

            best_iou = jnp.full((16,), -1.0, jnp.float32)
            best_a = jnp.zeros((16,), jnp.int32)
            aw_g = jnp.zeros((16,), jnp.float32)
            ah_g = jnp.zeros((16,), jnp.float32)
            for a, (aw, ah) in enumerate(_ANCHORS):
                inter = jnp.minimum(gt_w, aw) * jnp.minimum(gt_h, ah)
                union = gt_w * gt_h + aw * ah - inter
                iou_a = jnp.where(union > 0, inter / jnp.where(union > 0, union, 1.0), 0.0)
                upd = iou_a > best_iou
                best_iou = jnp.where(upd, iou_a, best_iou)
                best_a = jnp.where(upd, a, best_a)
                aw_g = jnp.where(upd, aw, aw_g)
                ah_g = jnp.where(upd, ah, ah_g)

            gi = jnp.clip((gx * float(_S)).astype(jnp.int32), 0, _S - 1)
            gj = jnp.clip((gy * float(_S)).astype(jnp.int32), 0, _S - 1)
            row169 = gj * _S + gi

            slot = (s * 2 + v) * 16 + iota16
            plsc.store_scatter(idxb_v, [slot], (_SPW * wid + s) * _CELLS + row169)

            cell = jnp.where(valid, best_a * _CELLS + row169, -1)
            sample.append(
                dict(valid=valid, slot=slot, best_a=best_a, gt_w=gt_w, gt_h=gt_h,
                     gx=gx, gy=gy, gi=gi, gj=gj, aw_g=aw_g, ah_g=ah_g, cell=cell)
            )

        per_vreg.append(sample)

    gather = pltpu.async_copy(tab_ref.at[idxb_v], rowsb_v, sem)

    for s in range(_SPW):
        sample = per_vreg[s]
        c0, c1 = sample[0]["cell"], sample[1]["cell"]
        cell_v[pl.ds(0, 16)] = c0
        cell_v[pl.ds(16, 16)] = c1
        dup0 = jnp.zeros((16,), jnp.bool_)
        dup1 = jnp.zeros((16,), jnp.bool_)
        for sh in range(1, 16):
            ids = jnp.minimum(iota16 + sh, 15)
            ok = iota16 + sh <= 15
            dup0 = dup0 | (ok & (plsc.load_gather(cell_v, [ids]) == c0))
            dup1 = dup1 | (ok & (plsc.load_gather(cell_v, [ids + 16]) == c1))
        for r in range(16):
            rot = plsc.load_gather(cell_v, [(iota16 + r) % 16 + 16])
            dup0 = dup0 | (rot == c0)
        sample[0]["kept"] = sample[0]["valid"] & jnp.logical_not(dup0)
        sample[1]["kept"] = sample[1]["valid"] & jnp.logical_not(dup1)

    gather.wait()

    acc_coord = jnp.zeros(16, jnp.float32)
    acc_obj = jnp.zeros(16, jnp.float32)
    acc_noobj = jnp.zeros(16, jnp.float32)
    for s in range(_SPW):
        for v in range(2):
            t = per_vreg[s][v]
            slot = t["slot"]
            col = t["best_a"] * 5

            tx = plsc.load_gather(rowsb_v, [slot, col + 0])
            ty = plsc.load_gather(rowsb_v, [slot, col + 1])
            tw = plsc.load_gather(rowsb_v, [slot, col + 2])
            th = plsc.load_gather(rowsb_v, [slot, col + 3])
            to = plsc.load_gather(rowsb_v, [slot, col + 4])
            pbx = _sigmoid(tx)
            pby = _sigmoid(ty)
            pbw = jnp.exp(tw) * t["aw_g"]
            pbh = jnp.exp(th) * t["ah_g"]
            conf_c = _sigmoid(to)

            gi_f = t["gi"].astype(jnp.float32)
            gj_f = t["gj"].astype(jnp.float32)
            gt_w, gt_h = t["gt_w"], t["gt_h"]
            cx_gt = t["gx"] * float(_S)
            cy_gt = t["gy"] * float(_S)
            ggx = cx_gt - gi_f
            ggy = cy_gt - gj_f
            cx_pr = pbx + gi_f
            cy_pr = pby + gj_f
            iw = jnp.maximum(
                0.0,
                jnp.minimum(cx_gt + gt_w * 0.5, cx_pr + pbw * 0.5)
                - jnp.maximum(cx_gt - gt_w * 0.5, cx_pr - pbw * 0.5),
            )
            ih = jnp.maximum(
                0.0,
                jnp.minimum(cy_gt + gt_h * 0.5, cy_pr + pbh * 0.5)
                - jnp.maximum(cy_gt - gt_h * 0.5, cy_pr - pbh * 0.5),
            )
            inter_a = iw * ih
            union_a = gt_w * gt_h + pbw * pbh - inter_a
            iou = jnp.where(union_a > 0, inter_a / jnp.where(union_a > 0, union_a, 1.0), 0.0)

            dx = pbx - ggx
            dy = pby - ggy
            dw = _sc_sqrt(pbw + 1e-6) - _sc_sqrt(gt_w + 1e-6)
            dh = _sc_sqrt(pbh + 1e-6) - _sc_sqrt(gt_h + 1e-6)
            coord_t = dx * dx + dy * dy + dw * dw + dh * dh
            do = iou - conf_c
            obj_t = do * do
            noobj_t = conf_c * conf_c

            kept = t["kept"]
            acc_coord = acc_coord + jnp.where(kept, coord_t, 0.0)
            acc_obj = acc_obj + jnp.where(kept, obj_t, 0.0)
            acc_noobj = acc_noobj + jnp.where(kept, noobj_t, 0.0)

    part16 = jnp.where(iota16 == 1, jnp.sum(acc_coord), 0.0)
    part16 = part16 + jnp.where(iota16 == 2, jnp.sum(acc_obj), 0.0)
    part16 = part16 + jnp.where(iota16 == 3, jnp.sum(acc_noobj), 0.0)
    obuf_v[...] = part16
    pltpu.sync_copy(obuf_v, shared_v.at[wid])
    plsc.subcore_barrier()

    @pl.when(wid == 0)
    def _final():
        pltpu.sync_copy(shared_v, allp_v)
        pltpu.sync_copy(sall_ref, sall_v)
        comb = jnp.zeros((16,), jnp.float32)
        for r in range(_NWK):
            comb = comb + allp_v[r, :]
        coord_raw = jnp.sum(jnp.where(iota16 == 1, comb, 0.0))
        obj_raw = jnp.sum(jnp.where(iota16 == 2, comb, 0.0))
        noobj_c = jnp.sum(jnp.where(iota16 == 3, comb, 0.0))
        s_all = jnp.sum(jnp.where(iota16 == 0, sall_v[0, :], 0.0))
        coord = _LC * coord_raw
        lnoobj = _LN * (s_all - noobj_c)
        tot = _LC * coord + obj_raw + _LN * lnoobj
        res = jnp.where(iota16 == 0, tot, 0.0)
        res = res + jnp.where(iota16 == 1, coord, 0.0)
        res = res + jnp.where(iota16 == 2, obj_raw + lnoobj, 0.0)
        obuf_v[...] = res
        pltpu.sync_copy(obuf_v, out_ref)


def kernel(predictions, targets, imgs):
    del imgs
    pred3 = predictions.reshape(_B, _CELLS, 425)
    tgt2 = targets.reshape(_B, _G * 5)
    wsel = jnp.asarray(_WSEL)

    scal = jax.ShapeDtypeStruct((1, 16), jnp.float32)
    table, s_all = pl.pallas_call(
        _prep_kernel,
        grid=(_B // _BPG,),
        in_specs=[
            pl.BlockSpec((_BPG, _CELLS, 425), lambda i: (i, 0, 0)),
            pl.BlockSpec((425, 128), lambda i: (0, 0)),
        ],
        out_specs=[
            pl.BlockSpec((_BPG * _CELLS, 128), lambda i: (i, 0)),
            pl.BlockSpec((1, 16), lambda i: (0, 0)),
        ],
        out_shape=[jax.ShapeDtypeStruct((_NROWS, 128), jnp.float32), scal],
    )(pred3, wsel)

    sc_fn = pl.kernel(
        _sc_body,
        out_type=jax.ShapeDtypeStruct((16,), jnp.float32),
        compiler_params=pltpu.CompilerParams(
            use_tc_tiling_on_sc=False, needs_layout_passes=False,
            skip_device_barrier=True,
        ),
        mesh=plsc.VectorSubcoreMesh(
            core_axis_name="c", subcore_axis_name="s", num_cores=1
        ),
        scratch_types=[
            pltpu.VMEM((_SPW, _G * 5), jnp.float32),
            pltpu.VMEM((128,), jnp.int32),
            pltpu.VMEM((128, 128), jnp.float32),
            pltpu.VMEM((32,), jnp.int32),
            pltpu.VMEM((16,), jnp.float32),
            pltpu.VMEM_SHARED((_NWK, 16), jnp.float32),
            pltpu.VMEM((_NWK, 16), jnp.float32),
            pltpu.VMEM((1, 16), jnp.float32),
            pltpu.SemaphoreType.DMA,
        ],
    )
    out = sc_fn(table, tgt2, s_all)
    return (out[0], out[1], out[2])


# speedup vs baseline: 1.0124x; 1.0002x over previous
import numpy as np
import jax
import jax.numpy as jnp
from jax import lax
from jax.experimental import pallas as pl
from jax.experimental.pallas import tpu as pltpu
from jax.experimental.pallas import tpu_sc as plsc

_LC = 5.0
_LN = 0.5
_ANCHORS = (
    (1.3221, 1.73145),
    (3.19275, 4.00944),
    (5.05587, 8.09892),
    (9.47112, 4.84053),
    (11.2364, 10.0071),
)
_A = 5
_S = 13
_G = 30
_B = 64
_NWK = 16
_SPW = _B // _NWK
_CELLS = _S * _S
_BPG = 16
_NROWS = _B * _CELLS


def _build_wsel():
    w = np.zeros((425, 128), np.float32)
    for a in range(_A):
        for k in range(5):
            w[85 * a + k, 5 * a + k] = 1.0
    return w


_WSEL = _build_wsel()


def _sigmoid(x):
    return 1.0 / (1.0 + jnp.exp(-x))


def _sc_sqrt(x):
    i = plsc.bitcast(x, jnp.int32)
    y = plsc.bitcast((i >> 1) + 0x1FBD1DF5, jnp.float32)
    for _ in range(3):
        y = 0.5 * (y + x / y)
    return y


def _prep_kernel(pred_ref, w_ref, tab_ref, sall_ref):
    i = pl.program_id(0)

    @pl.when(i == 0)
    def _init():
        sall_ref[...] = jnp.zeros((1, 16), jnp.float32)

    x = pred_ref[...].reshape(_BPG * _CELLS, 425)
    tab = jnp.dot(x, w_ref[...], preferred_element_type=jnp.float32)
    tab_ref[...] = tab

    col = jax.lax.broadcasted_iota(jnp.int32, (1, 128), 1)
    m = ((col < 5 * _A) & (col % 5 == 4)).astype(jnp.float32)
    conf = _sigmoid(tab)
    lane0 = (jax.lax.broadcasted_iota(jnp.int32, (1, 16), 1) == 0).astype(jnp.float32)
    sall_ref[...] += jnp.sum(conf * conf * m) * lane0


def _sc_body(tab_ref, tgt_ref, sall_ref, out_ref, tgt_v, idxb_v, rowsb_v, cell_v, obuf_v, shared_v, allp_v, sall_v, sem):
    wid = lax.axis_index("s")
    iota16 = lax.iota(jnp.int32, 16)

    pltpu.sync_copy(tgt_ref.at[pl.ds(_SPW * wid, _SPW)], tgt_v)

    per_vreg = []
    for s in range(_SPW):
        svec = jnp.full((16,), s, jnp.int32)
        sample = []
        for v in range(2):
            t16 = iota16 + 16 * v
            valid = t16 < _G
            tcl = jnp.minimum(t16, _G - 1)
            gx = plsc.load_gather(tgt_v, [svec, tcl * 5 + 0])
            gy = plsc.load_gather(tgt_v, [svec, tcl * 5 + 1])
            gw = plsc.load_gather(tgt_v, [svec, tcl * 5 + 2])
            gh = plsc.load_gather(tgt_v, [svec, tcl * 5 + 3])
            gt_w = gw * float(_S)
            gt_h = gh * float(_S)

            best_iou = jnp.full((16,), -1.0, jnp.float32)
            best_a = jnp.zeros((16,), jnp.int32)
            aw_g = jnp.zeros((16,), jnp.float32)
            ah_g = jnp.zeros((16,), jnp.float32)
            for a, (aw, ah) in enumerate(_ANCHORS):
                inter = jnp.minimum(gt_w, aw) * jnp.minimum(gt_h, ah)
                union = gt_w * gt_h + aw * ah - inter
                iou_a = jnp.where(union > 0, inter / jnp.where(union > 0, union, 1.0), 0.0)
                upd = iou_a > best_iou
                best_iou = jnp.where(upd, iou_a, best_iou)
                best_a = jnp.where(upd, a, best_a)
                aw_g = jnp.where(upd, aw, aw_g)
                ah_g = jnp.where(upd, ah, ah_g)

            gi = jnp.clip((gx * float(_S)).astype(jnp.int32), 0, _S - 1)
            gj = jnp.clip((gy * float(_S)).astype(jnp.int32), 0, _S - 1)
            row169 = gj * _S + gi

            slot = (s * 2 + v) * 16 + iota16
            plsc.store_scatter(idxb_v, [slot], (_SPW * wid + s) * _CELLS + row169)

            cell = jnp.where(valid, best_a * _CELLS + row169, -1)
            sample.append(
                dict(valid=valid, slot=slot, best_a=best_a, gt_w=gt_w, gt_h=gt_h,
                     gx=gx, gy=gy, gi=gi, gj=gj, aw_g=aw_g, ah_g=ah_g, cell=cell)
            )

        per_vreg.append(sample)

    gather = pltpu.async_copy(tab_ref.at[idxb_v], rowsb_v, sem)

    for s in range(_SPW):
        sample = per_vreg[s]
        c0, c1 = sample[0]["cell"], sample[1]["cell"]
        cell_v[pl.ds(0, 16)] = c0
        cell_v[pl.ds(16, 16)] = c1
        dup0 = jnp.zeros((16,), jnp.bool_)
        dup1 = jnp.zeros((16,), jnp.bool_)
        for sh in range(1, 16):
            ids = jnp.minimum(iota16 + sh, 15)
            ok = iota16 + sh <= 15
            dup0 = dup0 | (ok & (plsc.load_gather(cell_v, [ids]) == c0))
            dup1 = dup1 | (ok & (plsc.load_gather(cell_v, [ids + 16]) == c1))
        for r in range(16):
            rot = plsc.load_gather(cell_v, [(iota16 + r) % 16 + 16])
            dup0 = dup0 | (rot == c0)
        sample[0]["kept"] = sample[0]["valid"] & jnp.logical_not(dup0)
        sample[1]["kept"] = sample[1]["valid"] & jnp.logical_not(dup1)

    gather.wait()

    acc_coord = jnp.zeros(16, jnp.float32)
    acc_obj = jnp.zeros(16, jnp.float32)
    acc_noobj = jnp.zeros(16, jnp.float32)
    for s in range(_SPW):
        for v in range(2):
            t = per_vreg[s][v]
            slot = t["slot"]
            col = t["best_a"] * 5

            tx = plsc.load_gather(rowsb_v, [slot, col + 0])
            ty = plsc.load_gather(rowsb_v, [slot, col + 1])
            tw = plsc.load_gather(rowsb_v, [slot, col + 2])
            th = plsc.load_gather(rowsb_v, [slot, col + 3])
            to = plsc.load_gather(rowsb_v, [slot, col + 4])
            pbx = _sigmoid(tx)
            pby = _sigmoid(ty)
            pbw = jnp.exp(tw) * t["aw_g"]
            pbh = jnp.exp(th) * t["ah_g"]
            conf_c = _sigmoid(to)

            gi_f = t["gi"].astype(jnp.float32)
            gj_f = t["gj"].astype(jnp.float32)
            gt_w, gt_h = t["gt_w"], t["gt_h"]
            cx_gt = t["gx"] * float(_S)
            cy_gt = t["gy"] * float(_S)
            ggx = cx_gt - gi_f
            ggy = cy_gt - gj_f
            cx_pr = pbx + gi_f
            cy_pr = pby + gj_f
            iw = jnp.maximum(
                0.0,
                jnp.minimum(cx_gt + gt_w * 0.5, cx_pr + pbw * 0.5)
                - jnp.maximum(cx_gt - gt_w * 0.5, cx_pr - pbw * 0.5),
            )
            ih = jnp.maximum(
                0.0,
                jnp.minimum(cy_gt + gt_h * 0.5, cy_pr + pbh * 0.5)
                - jnp.maximum(cy_gt - gt_h * 0.5, cy_pr - pbh * 0.5),
            )
            inter_a = iw * ih
            union_a = gt_w * gt_h + pbw * pbh - inter_a
            iou = jnp.where(union_a > 0, inter_a / jnp.where(union_a > 0, union_a, 1.0), 0.0)

            dx = pbx - ggx
            dy = pby - ggy
            dw = _sc_sqrt(pbw + 1e-6) - _sc_sqrt(gt_w + 1e-6)
            dh = _sc_sqrt(pbh + 1e-6) - _sc_sqrt(gt_h + 1e-6)
            coord_t = dx * dx + dy * dy + dw * dw + dh * dh
            do = iou - conf_c
            obj_t = do * do
            noobj_t = conf_c * conf_c

            kept = t["kept"]
            acc_coord = acc_coord + jnp.where(kept, coord_t, 0.0)
            acc_obj = acc_obj + jnp.where(kept, obj_t, 0.0)
            acc_noobj = acc_noobj + jnp.where(kept, noobj_t, 0.0)

    part16 = jnp.where(iota16 == 1, jnp.sum(acc_coord), 0.0)
    part16 = part16 + jnp.where(iota16 == 2, jnp.sum(acc_obj), 0.0)
    part16 = part16 + jnp.where(iota16 == 3, jnp.sum(acc_noobj), 0.0)
    obuf_v[...] = part16
    pltpu.sync_copy(obuf_v, shared_v.at[wid])
    plsc.subcore_barrier()

    @pl.when(wid == 0)
    def _final():
        pltpu.sync_copy(shared_v, allp_v)
        pltpu.sync_copy(sall_ref, sall_v)
        comb = jnp.zeros((16,), jnp.float32)
        for r in range(_NWK):
            comb = comb + allp_v[r, :]
        coord_raw = jnp.sum(jnp.where(iota16 == 1, comb, 0.0))
        obj_raw = jnp.sum(jnp.where(iota16 == 2, comb, 0.0))
        noobj_c = jnp.sum(jnp.where(iota16 == 3, comb, 0.0))
        s_all = jnp.sum(jnp.where(iota16 == 0, sall_v[0, :], 0.0))
        coord = _LC * coord_raw
        lnoobj = _LN * (s_all - noobj_c)
        tot = _LC * coord + obj_raw + _LN * lnoobj
        res = jnp.where(iota16 == 0, tot, 0.0)
        res = res + jnp.where(iota16 == 1, coord, 0.0)
        res = res + jnp.where(iota16 == 2, obj_raw + lnoobj, 0.0)
        obuf_v[...] = res
        pltpu.sync_copy(obuf_v, out_ref)


def kernel(predictions, targets, imgs):
    del imgs
    pred3 = predictions.reshape(_B, _CELLS, 425)
    tgt2 = targets.reshape(_B, _G * 5)
    wsel = jnp.asarray(_WSEL)

    scal = jax.ShapeDtypeStruct((1, 16), jnp.float32)
    table, s_all = pl.pallas_call(
        _prep_kernel,
        grid=(_B // _BPG,),
        in_specs=[
            pl.BlockSpec((_BPG, _CELLS, 425), lambda i: (i, 0, 0)),
            pl.BlockSpec((425, 128), lambda i: (0, 0)),
        ],
        out_specs=[
            pl.BlockSpec((_BPG * _CELLS, 128), lambda i: (i, 0)),
            pl.BlockSpec((1, 16), lambda i: (0, 0)),
        ],
        out_shape=[jax.ShapeDtypeStruct((_NROWS, 128), jnp.float32), scal],
    )(pred3, wsel)

    sc_fn = pl.kernel(
        _sc_body,
        out_type=jax.ShapeDtypeStruct((16,), jnp.float32),
        compiler_params=pltpu.CompilerParams(
            use_tc_tiling_on_sc=False, needs_layout_passes=False,
            skip_device_barrier=True,
        ),
        mesh=plsc.VectorSubcoreMesh(
            core_axis_name="c", subcore_axis_name="s", num_cores=1
        ),
        scratch_types=[
            pltpu.VMEM((_SPW, _G * 5), jnp.float32),
            pltpu.VMEM((128,), jnp.int32),
            pltpu.VMEM((128, 128), jnp.float32),
            pltpu.VMEM((32,), jnp.int32),
            pltpu.VMEM((16,), jnp.float32),
            pltpu.VMEM_SHARED((_NWK, 16), jnp.float32),
            pltpu.VMEM((_NWK, 16), jnp.float32),
            pltpu.VMEM((1, 16), jnp.float32),
            pltpu.SemaphoreType.DMA,
        ],
    )
    out = sc_fn(table, tgt2, s_all)
    return (out[0], out[1], out[2])
